# baseline (speedup 1.0000x reference)
import functools

import jax
import jax.numpy as jnp
from jax import lax
from jax.experimental import pallas as pl
from jax.experimental.pallas import tpu as pltpu
from jax.experimental.pallas import tpu_sc as plsc

_NC = 2
_NS = 16
_NW = _NC * _NS
_CHUNK = 128


def kernel(x, embeddings):
    batch, hist = x.shape
    embed = embeddings.shape[1]
    num_idx = batch * hist
    assert num_idx % (8 * _NW) == 0
    b_per_w = num_idx // _NW
    assert b_per_w % _CHUNK == 0
    idx = x.reshape(num_idx)

    mesh = plsc.VectorSubcoreMesh(core_axis_name="c", subcore_axis_name="s")

    @functools.partial(
        pl.kernel,
        mesh=mesh,
        out_type=jax.ShapeDtypeStruct((num_idx, embed), embeddings.dtype),
        compiler_params=pltpu.CompilerParams(use_tc_tiling_on_sc=False),
        scratch_types=[
            pltpu.VMEM((b_per_w,), jnp.int32),
            pltpu.VMEM((_CHUNK, embed), jnp.float32),
            pltpu.SemaphoreType.DMA,
        ],
    )
    def gather_kernel(table_hbm, idx_hbm, out_hbm, idx_v, rows_v, sem):
        wid = lax.axis_index("s") * _NC + lax.axis_index("c")
        base = wid * b_per_w
        pltpu.sync_copy(idx_hbm.at[pl.ds(base, b_per_w)], idx_v)

        @pl.loop(0, b_per_w, step=_CHUNK)
        def _(c):
            pltpu.async_copy(
                table_hbm.at[idx_v.at[pl.ds(c, _CHUNK)]], rows_v, sem
            ).wait()
            pltpu.sync_copy(rows_v, out_hbm.at[pl.ds(base + c, _CHUNK)])

    out = gather_kernel(embeddings, idx)
    return out.reshape(batch, hist, embed)


# speedup vs baseline: 1.0462x; 1.0462x over previous
import functools

import jax
import jax.numpy as jnp
from jax import lax
from jax.experimental import pallas as pl
from jax.experimental.pallas import tpu as pltpu
from jax.experimental.pallas import tpu_sc as plsc

_NC = 2
_NS = 16
_NW = _NC * _NS
_W = 128
_NBUF = 10


def kernel(x, embeddings):
    batch, hist = x.shape
    embed = embeddings.shape[1]
    num_idx = batch * hist
    b_per_w = num_idx // _NW
    n_win = b_per_w // _W
    assert num_idx % (8 * _NW) == 0 and b_per_w % _W == 0 and n_win % _NBUF == 0
    idx = x.reshape(num_idx)

    mesh = plsc.VectorSubcoreMesh(core_axis_name="c", subcore_axis_name="s")

    @functools.partial(
        pl.kernel,
        mesh=mesh,
        out_type=jax.ShapeDtypeStruct((num_idx, embed), embeddings.dtype),
        compiler_params=pltpu.CompilerParams(use_tc_tiling_on_sc=False),
        scratch_types=[
            pltpu.VMEM((b_per_w,), jnp.int32),
            pltpu.VMEM((_NBUF, _W, embed), jnp.float32),
        ]
        + [pltpu.SemaphoreType.DMA] * (2 * _NBUF),
    )
    def gather_kernel(table_hbm, idx_hbm, out_hbm, idx_v, rows_v, *sems):
        gsem = sems[:_NBUF]
        ssem = sems[_NBUF:]
        wid = lax.axis_index("s") * _NC + lax.axis_index("c")
        base = wid * b_per_w
        pltpu.sync_copy(idx_hbm.at[pl.ds(base, b_per_w)], idx_v)

        def gather(c, b):
            return pltpu.make_async_copy(
                table_hbm.at[idx_v.at[pl.ds(c * _W, _W)]], rows_v.at[b], gsem[b]
            )

        def store(c, b):
            return pltpu.make_async_copy(
                rows_v.at[b], out_hbm.at[pl.ds(base + c * _W, _W)], ssem[b]
            )

        for b in range(_NBUF):
            gather(b, b).start()

        @pl.loop(0, n_win - _NBUF, step=_NBUF)
        def _(w0):
            for b in range(_NBUF):
                c = w0 + b
                gather(c, b).wait()
                store(c, b).start()
                store(c, b).wait()
                gather(c + _NBUF, b).start()

        for b in range(_NBUF):
            c = n_win - _NBUF + b
            gather(c, b).wait()
            store(c, b).start()
        for b in range(_NBUF):
            c = n_win - _NBUF + b
            store(c, b).wait()

    out = gather_kernel(embeddings, idx)
    return out.reshape(batch, hist, embed)


# speedup vs baseline: 1.4758x; 1.4105x over previous
import functools

import jax
import jax.numpy as jnp
from jax import lax
from jax.experimental import pallas as pl
from jax.experimental.pallas import tpu as pltpu
from jax.experimental.pallas import tpu_sc as plsc

_NC = 2
_NS = 16
_NW = _NC * _NS
_NR = 8192


def _widen_table(embeddings):
    vocab, embed = embeddings.shape
    nblk = pl.cdiv(vocab, 2 * _NR)
    t_view = embeddings.T

    def body(t_ref, w_ref):
        t = t_ref[...].T
        w_ref[:, :embed] = t[:_NR]
        w_ref[:, embed:] = t[_NR:]

    return pl.pallas_call(
        body,
        grid=(nblk,),
        in_specs=[pl.BlockSpec((embed, 2 * _NR), lambda i: (0, i))],
        out_specs=pl.BlockSpec((_NR, 2 * embed), lambda i: (i, 0)),
        out_shape=jax.ShapeDtypeStruct((nblk * _NR, 2 * embed), embeddings.dtype),
    )(t_view)


def kernel(x, embeddings):
    batch, hist = x.shape
    embed = embeddings.shape[1]
    bc = batch // _NW
    wide = _widen_table(embeddings)
    x_t = x.T

    mesh = plsc.VectorSubcoreMesh(core_axis_name="c", subcore_axis_name="s")

    @functools.partial(
        pl.kernel,
        mesh=mesh,
        out_type=jax.ShapeDtypeStruct((hist, embed, batch), embeddings.dtype),
        compiler_params=pltpu.CompilerParams(needs_layout_passes=False),
        scratch_types=[
            pltpu.VMEM((hist * bc,), jnp.int32),
            pltpu.VMEM((hist * bc,), jnp.int32),
            pltpu.VMEM((hist * bc,), jnp.int32),
            pltpu.VMEM((2, bc, 2 * embed), jnp.float32),
            pltpu.VMEM((2, embed, bc), jnp.float32),
        ]
        + [pltpu.SemaphoreType.DMA] * 5,
    )
    def sc_gather(wide_hbm, xt_hbm, out_hbm, idx_v, srow_v, hoff_v,
                  wbuf, obuf, gsem0, gsem1, ssem0, ssem1, xsem):
        gsem = (gsem0, gsem1)
        ssem = (ssem0, ssem1)
        wid = lax.axis_index("s") * _NC + lax.axis_index("c")
        b0 = wid * bc

        @pl.loop(0, hist)
        def _(h):
            pltpu.make_async_copy(
                xt_hbm.at[h, pl.ds(b0, bc)],
                idx_v.at[pl.ds(h * bc, bc)], xsem,
            ).start()

        @pl.loop(0, hist)
        def _(h):
            pltpu.make_async_copy(
                xt_hbm.at[h, pl.ds(b0, bc)],
                idx_v.at[pl.ds(h * bc, bc)], xsem,
            ).wait()

        @pl.loop(0, hist * bc, step=16)
        def _(k):
            v = idx_v[pl.ds(k, 16)]
            srow_v[pl.ds(k, 16)] = (
                lax.shift_right_logical(v, 14) * _NR + (v & (_NR - 1))
            )
            hoff_v[pl.ds(k, 16)] = (
                lax.shift_right_logical(v, 13) & 1
            ) * embed

        def gather(h, p):
            return pltpu.make_async_copy(
                wide_hbm.at[srow_v.at[pl.ds(h * bc, bc)]], wbuf.at[p], gsem[p]
            )

        def store(h, q):
            return pltpu.make_async_copy(
                obuf.at[q], out_hbm.at[h, :, pl.ds(b0, bc)], ssem[q]
            )

        def compact(h, p, q):
            @pl.loop(0, bc, step=16)
            def _(j):
                rows = j + lax.iota(jnp.int32, 16)
                hoff = hoff_v[pl.ds(h * bc + j, 16)]
                for e in range(embed):
                    vals = plsc.load_gather(wbuf.at[p], [rows, hoff + e])
                    obuf.at[q][e, pl.ds(j, 16)] = vals

        gather(0, 0).start()
        gather(0, 0).wait()
        gather(1, 1).start()
        compact(0, 0, 0)
        store(0, 0).start()
        gather(1, 1).wait()
        gather(2, 0).start()
        compact(1, 1, 1)
        store(1, 1).start()

        @pl.loop(2, hist - 2, step=2)
        def _(h):
            gather(h, 0).wait()
            gather(h + 1, 1).start()
            store(h, 0).wait()
            compact(h, 0, 0)
            store(h, 0).start()
            gather(h + 1, 1).wait()
            gather(h + 2, 0).start()
            store(h + 1, 1).wait()
            compact(h + 1, 1, 1)
            store(h + 1, 1).start()

        gather(hist - 2, 0).wait()
        gather(hist - 1, 1).start()
        store(hist - 2, 0).wait()
        compact(hist - 2, 0, 0)
        store(hist - 2, 0).start()
        gather(hist - 1, 1).wait()
        store(hist - 1, 1).wait()
        compact(hist - 1, 1, 1)
        store(hist - 1, 1).start()
        store(hist - 2, 0).wait()
        store(hist - 1, 1).wait()

    out_t = sc_gather(wide, x_t)
    return jnp.transpose(out_t, (2, 0, 1))


# speedup vs baseline: 1.9162x; 1.2984x over previous
import functools

import jax
import jax.numpy as jnp
from jax import lax
from jax.experimental import pallas as pl
from jax.experimental.pallas import tpu as pltpu
from jax.experimental.pallas import tpu_sc as plsc

_NC = 2
_NS = 16
_NW = _NC * _NS
_NR = 8192


def _widen_table(embeddings):
    vocab, embed = embeddings.shape
    nblk = pl.cdiv(vocab, 2 * _NR)
    t_view = embeddings.T

    def body(t_ref, w_ref):
        t = t_ref[...].T
        w_ref[:, :embed] = t[:_NR]
        w_ref[:, embed:] = t[_NR:]

    return pl.pallas_call(
        body,
        grid=(nblk,),
        in_specs=[pl.BlockSpec((embed, 2 * _NR), lambda i: (0, i))],
        out_specs=pl.BlockSpec((_NR, 2 * embed), lambda i: (i, 0)),
        out_shape=jax.ShapeDtypeStruct((nblk * _NR, 2 * embed), embeddings.dtype),
        compiler_params=pltpu.CompilerParams(
            dimension_semantics=("parallel",)
        ),
    )(t_view)


def kernel(x, embeddings):
    batch, hist = x.shape
    embed = embeddings.shape[1]
    bc = batch // _NW
    wide = _widen_table(embeddings)
    x_t = x.T

    mesh = plsc.VectorSubcoreMesh(core_axis_name="c", subcore_axis_name="s")

    @functools.partial(
        pl.kernel,
        mesh=mesh,
        out_type=jax.ShapeDtypeStruct((hist, embed, batch), embeddings.dtype),
        compiler_params=pltpu.CompilerParams(needs_layout_passes=False),
        scratch_types=[
            pltpu.VMEM((hist * bc,), jnp.int32),
            pltpu.VMEM((hist * bc,), jnp.int32),
            pltpu.VMEM((hist * bc,), jnp.int32),
            pltpu.VMEM((2, bc, 2 * embed), jnp.float32),
            pltpu.VMEM((2, embed, bc), jnp.float32),
        ]
        + [pltpu.SemaphoreType.DMA] * 5,
    )
    def sc_gather(wide_hbm, xt_hbm, out_hbm, idx_v, srow_v, hoff_v,
                  wbuf, obuf, gsem0, gsem1, ssem0, ssem1, xsem):
        gsem = (gsem0, gsem1)
        ssem = (ssem0, ssem1)
        wid = lax.axis_index("s") * _NC + lax.axis_index("c")
        b0 = wid * bc

        @pl.loop(0, hist)
        def _(h):
            pltpu.make_async_copy(
                xt_hbm.at[h, pl.ds(b0, bc)],
                idx_v.at[pl.ds(h * bc, bc)], xsem,
            ).start()

        @pl.loop(0, hist)
        def _(h):
            pltpu.make_async_copy(
                xt_hbm.at[h, pl.ds(b0, bc)],
                idx_v.at[pl.ds(h * bc, bc)], xsem,
            ).wait()

        @pl.loop(0, hist * bc, step=16)
        def _(k):
            v = idx_v[pl.ds(k, 16)]
            srow_v[pl.ds(k, 16)] = (
                lax.shift_right_logical(v, 14) * _NR + (v & (_NR - 1))
            )
            hoff_v[pl.ds(k, 16)] = (
                lax.shift_right_logical(v, 13) & 1
            ) * embed

        def gather(h, p):
            return pltpu.make_async_copy(
                wide_hbm.at[srow_v.at[pl.ds(h * bc, bc)]], wbuf.at[p], gsem[p]
            )

        def store(h, q):
            return pltpu.make_async_copy(
                obuf.at[q], out_hbm.at[h, :, pl.ds(b0, bc)], ssem[q]
            )

        def compact(h, p, q):
            @pl.loop(0, bc, step=16)
            def _(j):
                rows = j + lax.iota(jnp.int32, 16)
                hoff = hoff_v[pl.ds(h * bc + j, 16)]
                for e0 in range(0, embed, 8):
                    vals = [
                        plsc.load_gather(wbuf.at[p], [rows, hoff + (e0 + u)])
                        for u in range(8)
                    ]
                    for u in range(8):
                        obuf.at[q][e0 + u, pl.ds(j, 16)] = vals[u]

        gather(0, 0).start()
        gather(0, 0).wait()
        gather(1, 1).start()
        compact(0, 0, 0)
        store(0, 0).start()
        gather(1, 1).wait()
        gather(2, 0).start()
        compact(1, 1, 1)
        store(1, 1).start()

        @pl.loop(2, hist - 2, step=2)
        def _(h):
            gather(h, 0).wait()
            gather(h + 1, 1).start()
            store(h, 0).wait()
            compact(h, 0, 0)
            store(h, 0).start()
            gather(h + 1, 1).wait()
            gather(h + 2, 0).start()
            store(h + 1, 1).wait()
            compact(h + 1, 1, 1)
            store(h + 1, 1).start()

        gather(hist - 2, 0).wait()
        gather(hist - 1, 1).start()
        store(hist - 2, 0).wait()
        compact(hist - 2, 0, 0)
        store(hist - 2, 0).start()
        gather(hist - 1, 1).wait()
        store(hist - 1, 1).wait()
        compact(hist - 1, 1, 1)
        store(hist - 1, 1).start()
        store(hist - 2, 0).wait()
        store(hist - 1, 1).wait()

    out_t = sc_gather(wide, x_t)
    return jnp.transpose(out_t, (2, 0, 1))


# speedup vs baseline: 1.9272x; 1.0058x over previous
import functools

import jax
import jax.numpy as jnp
from jax import lax
from jax.experimental import pallas as pl
from jax.experimental.pallas import tpu as pltpu
from jax.experimental.pallas import tpu_sc as plsc

_NC = 2
_NS = 16
_NW = _NC * _NS
_NR = 8192


def _widen_table(embeddings):
    vocab, embed = embeddings.shape
    nblk = pl.cdiv(vocab, 2 * _NR)
    t_view = embeddings.T

    def body(t_ref, w_ref):
        t = t_ref[...].T
        w_ref[:, :embed] = t[:_NR]
        w_ref[:, embed:] = t[_NR:]

    return pl.pallas_call(
        body,
        grid=(nblk,),
        in_specs=[pl.BlockSpec((embed, 2 * _NR), lambda i: (0, i))],
        out_specs=pl.BlockSpec((_NR, 2 * embed), lambda i: (i, 0)),
        out_shape=jax.ShapeDtypeStruct((nblk * _NR, 2 * embed), embeddings.dtype),
        compiler_params=pltpu.CompilerParams(
            dimension_semantics=("parallel",)
        ),
    )(t_view)


def kernel(x, embeddings):
    batch, hist = x.shape
    embed = embeddings.shape[1]
    bc = batch // _NW
    wide = _widen_table(embeddings)
    x_t = x.T

    mesh = plsc.VectorSubcoreMesh(core_axis_name="c", subcore_axis_name="s")

    @functools.partial(
        pl.kernel,
        mesh=mesh,
        out_type=jax.ShapeDtypeStruct((hist, embed, batch), embeddings.dtype),
        compiler_params=pltpu.CompilerParams(needs_layout_passes=False),
        scratch_types=[
            pltpu.VMEM((hist * bc,), jnp.int32),
            pltpu.VMEM((hist * bc,), jnp.int32),
            pltpu.VMEM((hist * bc,), jnp.int32),
            pltpu.VMEM((2, bc, 2 * embed), jnp.float32),
            pltpu.VMEM((2, embed, bc), jnp.float32),
        ]
        + [pltpu.SemaphoreType.DMA] * 5,
    )
    def sc_gather(wide_hbm, xt_hbm, out_hbm, idx_v, srow_v, hoff_v,
                  wbuf, obuf, gsem0, gsem1, ssem0, ssem1, xsem):
        gsem = (gsem0, gsem1)
        ssem = (ssem0, ssem1)
        wid = lax.axis_index("s") * _NC + lax.axis_index("c")
        b0 = wid * bc

        @pl.loop(0, hist)
        def _(h):
            pltpu.make_async_copy(
                xt_hbm.at[h, pl.ds(b0, bc)],
                idx_v.at[pl.ds(h * bc, bc)], xsem,
            ).start()

        @pl.loop(0, hist)
        def _(h):
            pltpu.make_async_copy(
                xt_hbm.at[h, pl.ds(b0, bc)],
                idx_v.at[pl.ds(h * bc, bc)], xsem,
            ).wait()

        @pl.loop(0, hist * bc, step=16)
        def _(k):
            v = idx_v[pl.ds(k, 16)]
            srow_v[pl.ds(k, 16)] = (
                lax.shift_right_logical(v, 14) * _NR + (v & (_NR - 1))
            )
            hoff_v[pl.ds(k, 16)] = (
                lax.shift_right_logical(v, 13) & 1
            ) * embed

        def gather(h, p):
            return pltpu.make_async_copy(
                wide_hbm.at[srow_v.at[pl.ds(h * bc, bc)]], wbuf.at[p], gsem[p]
            )

        def store(h, q):
            return pltpu.make_async_copy(
                obuf.at[q], out_hbm.at[h, :, pl.ds(b0, bc)], ssem[q]
            )

        def compact(h, p, q):
            @pl.loop(0, bc, step=16)
            def _(j):
                rows = j + lax.iota(jnp.int32, 16)
                hoff = hoff_v[pl.ds(h * bc + j, 16)]
                for e0 in range(0, embed, 16):
                    vals = [
                        plsc.load_gather(wbuf.at[p], [rows, hoff + (e0 + u)])
                        for u in range(16)
                    ]
                    for u in range(16):
                        obuf.at[q][e0 + u, pl.ds(j, 16)] = vals[u]

        gather(0, 0).start()
        gather(0, 0).wait()
        gather(1, 1).start()
        compact(0, 0, 0)
        store(0, 0).start()
        gather(1, 1).wait()
        gather(2, 0).start()
        compact(1, 1, 1)
        store(1, 1).start()

        @pl.loop(2, hist - 2, step=2)
        def _(h):
            gather(h, 0).wait()
            gather(h + 1, 1).start()
            store(h, 0).wait()
            compact(h, 0, 0)
            store(h, 0).start()
            gather(h + 1, 1).wait()
            gather(h + 2, 0).start()
            store(h + 1, 1).wait()
            compact(h + 1, 1, 1)
            store(h + 1, 1).start()

        gather(hist - 2, 0).wait()
        gather(hist - 1, 1).start()
        store(hist - 2, 0).wait()
        compact(hist - 2, 0, 0)
        store(hist - 2, 0).start()
        gather(hist - 1, 1).wait()
        store(hist - 1, 1).wait()
        compact(hist - 1, 1, 1)
        store(hist - 1, 1).start()
        store(hist - 2, 0).wait()
        store(hist - 1, 1).wait()

    out_t = sc_gather(wide, x_t)
    return jnp.transpose(out_t, (2, 0, 1))
